# baseline (speedup 1.0000x reference)
import functools

import jax
import jax.numpy as jnp
from jax import lax
from jax.experimental import pallas as pl
from jax.experimental.pallas import tpu as pltpu
from jax.experimental.pallas import tpu_sc as plsc

_NUM_CORES = 2
_NUM_SUBCORES = 16
_NW = _NUM_CORES * _NUM_SUBCORES
_LANES = 16

_CHUNKS = ((0, 200),)
_NBUF = 4


def _avg_pool_sc(idx_flat, table_pk, batch, hist, dim):
    spw = batch // _NW
    sblk = 128
    nblk = spw // sblk
    inv_l = float(1.0 / hist)
    half = dim // 2

    mesh = plsc.VectorSubcoreMesh(core_axis_name="c", subcore_axis_name="s")

    @functools.partial(
        pl.kernel,
        mesh=mesh,
        compiler_params=pltpu.CompilerParams(
            use_tc_tiling_on_sc=False, needs_layout_passes=False),
        out_type=jax.ShapeDtypeStruct((batch, dim), jnp.float32),
        scratch_types=(
            [pltpu.VMEM((sblk * hist,), jnp.int32)]
            + [pltpu.VMEM((hist, half), jnp.int32)
               for _ in range(_NBUF)]
            + [pltpu.VMEM((spw, dim), jnp.float32)]
            + [pltpu.SemaphoreType.DMA for _ in range(_NBUF)]
        ),
    )
    def k(idx_hbm, table_hbm, out_hbm, idx_v, *rest):
        rows = rest[:_NBUF]
        out_v = rest[_NBUF]
        sems = rest[_NBUF + 1:]
        wid = lax.axis_index("s") * _NUM_CORES + lax.axis_index("c")
        base = wid * spw

        def gather_start(rows, sem, s_in_blk):
            off = s_in_blk * hist
            for c0, cn in _CHUNKS:
                pltpu.make_async_copy(
                    table_hbm.at[idx_v.at[pl.ds(off + c0, cn)]],
                    rows.at[pl.ds(c0, cn), :],
                    sem,
                ).start()

        def gather_wait(rows, sem):
            for c0, cn in _CHUNKS:
                pltpu.make_async_copy(
                    table_hbm.at[idx_v.at[pl.ds(c0, cn)]],
                    rows.at[pl.ds(c0, cn), :],
                    sem,
                ).wait()

        def reduce_sample(rows, s_out):
            zero = jnp.zeros((_LANES,), jnp.float32)
            mask = jnp.int32(-65536)

            def tree8(vals):
                s01, s23 = vals[0] + vals[1], vals[2] + vals[3]
                s45, s67 = vals[4] + vals[5], vals[6] + vals[7]
                return (s01 + s23) + (s45 + s67)

            def body(i, accs):
                ev, od = accs
                r = i * 8
                x = [rows[r + j, pl.ds(0, half)] for j in range(8)]
                lo = [plsc.bitcast(v << 16, jnp.float32) for v in x]
                hi = [plsc.bitcast(v & mask, jnp.float32) for v in x]
                return ev + tree8(lo), od + tree8(hi)

            ev, od = lax.fori_loop(0, hist // 8, body, (zero, zero))
            out_v[s_out, pl.ds(0, _LANES)] = ev * inv_l
            out_v[s_out, pl.ds(_LANES, _LANES)] = od * inv_l

        @pl.loop(0, nblk)
        def _blk(blk):
            blk_sample = base + blk * sblk
            pltpu.sync_copy(
                idx_hbm.at[pl.ds(blk_sample * hist, sblk * hist)], idx_v)
            for j in range(_NBUF - 1):
                gather_start(rows[j], sems[j], j)

            @pl.loop(0, sblk, step=_NBUF)
            def _s(s0):
                for j in range(_NBUF):
                    s = s0 + j
                    gather_wait(rows[j], sems[j])
                    reduce_sample(rows[j], blk * sblk + s)
                    jn = (j + _NBUF - 1) % _NBUF

                    @pl.when(s + _NBUF - 1 < sblk)
                    def _():
                        gather_start(rows[jn], sems[jn], s + _NBUF - 1)

        pltpu.sync_copy(out_v, out_hbm.at[pl.ds(base, spw), :])

    return k(idx_flat, table_pk)


def kernel(instances, table):
    batch, hist = instances.shape
    nv, dim = table.shape
    idx_flat = instances.astype(jnp.int32).reshape(batch * hist)
    table_pk = jax.lax.bitcast_convert_type(
        table.astype(jnp.bfloat16).reshape(nv, dim // 2, 2), jnp.int32)
    out_pk = _avg_pool_sc(idx_flat, table_pk, batch, hist, dim)
    return (out_pk.reshape(batch, 2, dim // 2)
            .transpose(0, 2, 1).reshape(batch, dim))


# speedup vs baseline: 1.7191x; 1.7191x over previous
import functools

import jax
import jax.numpy as jnp
from jax import lax
from jax.experimental import pallas as pl
from jax.experimental.pallas import tpu as pltpu
from jax.experimental.pallas import tpu_sc as plsc

_NUM_CORES = 2
_NUM_SUBCORES = 16
_NW = _NUM_CORES * _NUM_SUBCORES
_LANES = 16
_NBUF = 4


def _avg_pool_sc(instances, table, batch, hist, dim):
    spw = batch // _NW
    sblk = 128
    nblk = spw // sblk
    inv_l = float(1.0 / hist)

    mesh = plsc.VectorSubcoreMesh(core_axis_name="c", subcore_axis_name="s")

    @functools.partial(
        pl.kernel,
        mesh=mesh,
        compiler_params=pltpu.CompilerParams(use_tc_tiling_on_sc=False),
        out_type=jax.ShapeDtypeStruct((batch, dim), jnp.float32),
        scratch_types=(
            [pltpu.VMEM((sblk, hist), jnp.int32)]
            + [pltpu.VMEM((hist, dim), jnp.float32)
               for _ in range(_NBUF)]
            + [pltpu.VMEM((spw, dim), jnp.float32)]
            + [pltpu.SemaphoreType.DMA for _ in range(_NBUF)]
        ),
    )
    def k(idx_hbm, table_hbm, out_hbm, idx_v, *rest):
        rows = rest[:_NBUF]
        out_v = rest[_NBUF]
        sems = rest[_NBUF + 1:]
        wid = lax.axis_index("s") * _NUM_CORES + lax.axis_index("c")
        base = wid * spw

        def gather_start(buf, sem, s_in_blk):
            pltpu.make_async_copy(
                table_hbm.at[idx_v.at[s_in_blk]], buf, sem).start()

        def gather_wait(buf, sem):
            pltpu.make_async_copy(
                table_hbm.at[idx_v.at[0]], buf, sem).wait()

        def reduce_sample(buf, s_out):
            zero = jnp.zeros((_LANES,), jnp.float32)

            def tree8(vals):
                s01, s23 = vals[0] + vals[1], vals[2] + vals[3]
                s45, s67 = vals[4] + vals[5], vals[6] + vals[7]
                return (s01 + s23) + (s45 + s67)

            def body(i, accs):
                lo, hi = accs
                r = i * 8
                tl = [buf[r + j, pl.ds(0, _LANES)] for j in range(8)]
                th = [buf[r + j, pl.ds(_LANES, _LANES)] for j in range(8)]
                return lo + tree8(tl), hi + tree8(th)

            lo, hi = lax.fori_loop(0, hist // 8, body, (zero, zero))
            out_v[s_out, pl.ds(0, _LANES)] = lo * inv_l
            out_v[s_out, pl.ds(_LANES, _LANES)] = hi * inv_l

        @pl.loop(0, nblk)
        def _blk(blk):
            blk_sample = base + blk * sblk
            pltpu.sync_copy(
                idx_hbm.at[pl.ds(blk_sample, sblk), :], idx_v)
            for j in range(_NBUF - 1):
                gather_start(rows[j], sems[j], j)

            @pl.loop(0, sblk, step=_NBUF)
            def _s(s0):
                for j in range(_NBUF):
                    s = s0 + j
                    gather_wait(rows[j], sems[j])
                    reduce_sample(rows[j], blk * sblk + s)
                    jn = (j + _NBUF - 1) % _NBUF

                    @pl.when(s + _NBUF - 1 < sblk)
                    def _():
                        gather_start(rows[jn], sems[jn], s + _NBUF - 1)

        pltpu.sync_copy(out_v, out_hbm.at[pl.ds(base, spw), :])

    return k(instances, table)


def kernel(instances, table):
    batch, hist = instances.shape
    _, dim = table.shape
    return _avg_pool_sc(instances.astype(jnp.int32), table, batch, hist, dim)
